# baseline (speedup 1.0000x reference)
import functools

import jax
import jax.numpy as jnp
from jax import lax
from jax.experimental import pallas as pl
from jax.experimental.pallas import tpu as pltpu
from jax.experimental.pallas import tpu_sc as plsc

EPS = 1e-12
NC = 2
NS = 16
L = 16
C = 32
NB = 32
EP = 327680
NP = 10240

_MESH = plsc.VectorSubcoreMesh(core_axis_name="c", subcore_axis_name="s",
                               num_cores=NC)


def _make_deg_kernel(deg_pad):
    nch_tile = EP // C // NS
    zrows = deg_pad // NS

    @functools.partial(
        pl.kernel,
        out_type=jax.ShapeDtypeStruct((NC, 1, deg_pad), jnp.float32),
        mesh=_MESH,
        scratch_types=[
            pltpu.VMEM((NB, C), jnp.int32),
            pltpu.VMEM((NB, C), jnp.float32),
            pltpu.VMEM((zrows,), jnp.float32),
            pltpu.VMEM_SHARED((deg_pad,), jnp.float32),
        ],
    )
    def deg_kernel(src2, dst2, w2, out, rows_v, w_v, zero_v, deg_sp):
        c = lax.axis_index("c")
        s = lax.axis_index("s")

        def zb(i, _):
            zero_v[pl.ds(i * L, L)] = jnp.zeros((L,), jnp.float32)
            return 0
        lax.fori_loop(0, zrows // L, zb, 0)
        pltpu.sync_copy(zero_v, deg_sp.at[pl.ds(s * zrows, zrows)])
        plsc.subcore_barrier()

        base = s * nch_tile

        def block(b, _):
            rb = base + b * NB
            pltpu.sync_copy(w2.at[pl.ds(rb, NB)], w_v)

            @pl.when(c == 0)
            def _():
                pltpu.sync_copy(src2.at[pl.ds(rb, NB)], rows_v)

            @pl.when(c != 0)
            def _():
                pltpu.sync_copy(dst2.at[pl.ds(rb, NB)], rows_v)

            def chunk(j, _):
                pltpu.sync_copy(w_v.at[j], deg_sp.at[rows_v.at[j]], add=True)
                return 0
            lax.fori_loop(0, NB, chunk, 0)
            return 0
        lax.fori_loop(0, nch_tile // NB, block, 0)

        plsc.subcore_barrier()
        pltpu.sync_copy(deg_sp.at[pl.ds(s * zrows, zrows)],
                        out.at[c, 0, pl.ds(s * zrows, zrows)])

    return deg_kernel


def _make_spmm_kernel(D, pass2):
    nch_tile = (2 * EP if pass2 else EP) // C // NS
    rows_out = NP // NS
    BLK = 64
    nblocks = nch_tile // BLK

    @functools.partial(
        pl.kernel,
        out_type=jax.ShapeDtypeStruct((NC, NP, D), jnp.float32),
        mesh=_MESH,
        scratch_types=[
            pltpu.VMEM((BLK, C), jnp.int32),
            pltpu.VMEM((BLK, C), jnp.int32),
            pltpu.VMEM((BLK, C), jnp.float32),
            pltpu.VMEM((C, D), jnp.float32),
            pltpu.VMEM((C, D), jnp.float32),
            pltpu.VMEM((C, D), jnp.float32),
            pltpu.VMEM((C, D), jnp.float32),
            pltpu.SemaphoreType.DMA,
            pltpu.SemaphoreType.DMA,
            pltpu.SemaphoreType.DMA,
            pltpu.SemaphoreType.DMA,
            pltpu.VMEM_SHARED((NP, D), jnp.float32),
        ],
    )
    def spmm_kernel(x2, src2, dst2, w2, out,
                    rows_v, cols_v, w_v, g0, g1, s0, s1,
                    gsem0, gsem1, ssem0, ssem1, acc_sp):
        c = lax.axis_index("c")
        s = lax.axis_index("s")
        G = (g0, g1)
        S = (s0, s1)
        GSEM = (gsem0, gsem1)
        SSEM = (ssem0, ssem1)
        if pass2:
            xv = x2.at[c]
        else:
            xv = x2
        dummy = xv.at[pl.ds(0, C)]

        def zrow(e, _):
            for k in range(D // L):
                s0[e, pl.ds(k * L, L)] = jnp.zeros((L,), jnp.float32)
            return 0
        lax.fori_loop(0, C, zrow, 0)
        for m in range(rows_out // C):
            pltpu.sync_copy(s0, acc_sp.at[pl.ds(s * rows_out + m * C, C)])
        plsc.subcore_barrier()

        if pass2:
            d = s // (NS // 2)
            base = (s % (NS // 2)) * nch_tile
        else:
            d = c
            base = s * nch_tile

        def fire_gather(j, b):
            pltpu.async_copy(xv.at[cols_v.at[j]], G[b], GSEM[b])

        for blk in range(nblocks):
            rb = base + blk * BLK
            pltpu.sync_copy(w2.at[pl.ds(rb, BLK)], w_v)

            @pl.when(d == 0)
            def _():
                pltpu.sync_copy(src2.at[pl.ds(rb, BLK)], rows_v)
                pltpu.sync_copy(dst2.at[pl.ds(rb, BLK)], cols_v)

            @pl.when(d != 0)
            def _():
                pltpu.sync_copy(dst2.at[pl.ds(rb, BLK)], rows_v)
                pltpu.sync_copy(src2.at[pl.ds(rb, BLK)], cols_v)

            fire_gather(0, 0)
            fire_gather(1, 1)

            def pair(i, _):
                for b in range(2):
                    j = i * 2 + b
                    pltpu.make_async_copy(dummy, G[b], GSEM[b]).wait()

                    def sgroup(gi, _):
                        wvec = w_v[j, pl.ds(gi * L, L)]
                        for lane in range(L):
                            wv = jnp.full((L,), wvec[lane], jnp.float32)
                            e = gi * L + lane
                            for k in range(D // L):
                                sl = pl.ds(k * L, L)
                                S[b][e, sl] = G[b][e, sl] * wv
                        return 0
                    lax.fori_loop(0, C // L, sgroup, 0)

                    @pl.when(j + 2 < BLK)
                    def _():
                        fire_gather(j + 2, b)
                return 0
            lax.fori_loop(0, BLK // 2, pair, 0)

        plsc.subcore_barrier()
        pltpu.sync_copy(acc_sp.at[pl.ds(s * rows_out, rows_out)],
                        out.at[c, pl.ds(s * rows_out, rows_out)])

    return spmm_kernel


def _edge_mlp_body(ea_ref, we1_ref, be1_ref, we2_ref, be2_ref, o_ref):
    eh = jnp.maximum(
        jnp.dot(ea_ref[...], we1_ref[...],
                preferred_element_type=jnp.float32) + be1_ref[...], 0.0)
    logit = jnp.dot(eh, we2_ref[...],
                    preferred_element_type=jnp.float32) + be2_ref[0]
    o_ref[...] = jax.nn.sigmoid(logit)


def _x_body(h_ref, w1_ref, w2_ref, degt_ref, x1_ref, xs2_ref):
    degt = degt_ref[...]
    dis = lax.rsqrt(degt[:, 0] + degt[:, 1] + EPS)
    x1 = jnp.dot(h_ref[...], w1_ref[...], preferred_element_type=jnp.float32)
    x2 = jnp.dot(h_ref[...], w2_ref[...], preferred_element_type=jnp.float32)
    x1_ref[...] = x1
    xs2_ref[...] = x2 * dis[:, None]


def _mid_body(x1_ref, u_ref, degt_ref, o_ref):
    degt = degt_ref[...]
    deg = degt[:, 0] + degt[:, 1] + EPS
    dis = lax.rsqrt(deg)
    u = u_ref[0] + u_ref[1]
    o_ref[0] = x1_ref[...] * dis[:, None]
    o_ref[1] = u * (1.0 / deg)[:, None]


def _final_body(tv_ref, h_ref, degt_ref, g_ref, wua_ref, wuh_ref, bu_ref,
                o_ref):
    gc = jnp.clip(g_ref[0], 0.0, 1.0)
    degt = degt_ref[...]
    dis = lax.rsqrt(degt[:, 0] + degt[:, 1] + EPS)
    agg = (gc * tv_ref[0] + (1.0 - gc) * tv_ref[1]) * dis[:, None]
    acc = jnp.dot(agg, wua_ref[...], preferred_element_type=jnp.float32)
    acc += jnp.dot(h_ref[...], wuh_ref[...],
                   preferred_element_type=jnp.float32)
    o_ref[...] = jnp.maximum(acc + bu_ref[...], 0.0)


def kernel(H, edge_index, edge_attr, W_self, b_self, W1, W2, We1, be1, We2,
           be2, g, Wu, bu):
    N, D_IN = H.shape
    E, D_EDGE = edge_attr.shape
    D_OUT = Wu.shape[1]
    H_EDGE = We1.shape[1]
    deg_pad = NP

    BE = 8000
    edge_w = pl.pallas_call(
        _edge_mlp_body,
        out_shape=jax.ShapeDtypeStruct((E, 1), jnp.float32),
        grid=(E // BE,),
        in_specs=[
            pl.BlockSpec((BE, D_EDGE), lambda i: (i, 0)),
            pl.BlockSpec((D_EDGE, H_EDGE), lambda i: (0, 0)),
            pl.BlockSpec((H_EDGE,), lambda i: (0,)),
            pl.BlockSpec((H_EDGE, 1), lambda i: (0, 0)),
            pl.BlockSpec((1,), lambda i: (0,)),
        ],
        out_specs=pl.BlockSpec((BE, 1), lambda i: (i, 0)),
    )(edge_attr, We1, be1, We2, be2)

    pad = EP - E
    src2 = jnp.concatenate([edge_index[0],
                            jnp.zeros((pad,), jnp.int32)]).reshape(EP // C, C)
    dst2 = jnp.concatenate([edge_index[1],
                            jnp.zeros((pad,), jnp.int32)]).reshape(EP // C, C)
    w2 = jnp.concatenate([edge_w[:, 0],
                          jnp.zeros((pad,), jnp.float32)]).reshape(EP // C, C)

    degp = _make_deg_kernel(deg_pad)(src2, dst2, w2)
    degt = degp[:, 0, :N].T

    BN = 1000
    x1, xs2 = pl.pallas_call(
        _x_body,
        out_shape=[jax.ShapeDtypeStruct((N, D_OUT), jnp.float32),
                   jax.ShapeDtypeStruct((N, D_OUT), jnp.float32)],
        grid=(N // BN,),
        in_specs=[
            pl.BlockSpec((BN, D_IN), lambda i: (i, 0)),
            pl.BlockSpec((D_IN, D_OUT), lambda i: (0, 0)),
            pl.BlockSpec((D_IN, D_OUT), lambda i: (0, 0)),
            pl.BlockSpec((BN, 2), lambda i: (i, 0)),
        ],
        out_specs=[pl.BlockSpec((BN, D_OUT), lambda i: (i, 0)),
                   pl.BlockSpec((BN, D_OUT), lambda i: (i, 0))],
    )(H, W1, W2, degt)

    up = _make_spmm_kernel(D_OUT, pass2=False)(xs2, src2, dst2, w2)

    xtv = pl.pallas_call(
        _mid_body,
        out_shape=jax.ShapeDtypeStruct((2, NP, D_OUT), jnp.float32),
        grid=(N // BN,),
        in_specs=[
            pl.BlockSpec((BN, D_OUT), lambda i: (i, 0)),
            pl.BlockSpec((2, BN, D_OUT), lambda i: (0, i, 0)),
            pl.BlockSpec((BN, 2), lambda i: (i, 0)),
        ],
        out_specs=pl.BlockSpec((2, BN, D_OUT), lambda i: (0, i, 0)),
    )(x1, up, degt)

    tv = _make_spmm_kernel(D_OUT, pass2=True)(xtv, src2, dst2, w2)

    out = pl.pallas_call(
        _final_body,
        out_shape=jax.ShapeDtypeStruct((N, D_OUT), jnp.float32),
        grid=(N // BN,),
        in_specs=[
            pl.BlockSpec((2, BN, D_OUT), lambda i: (0, i, 0)),
            pl.BlockSpec((BN, D_IN), lambda i: (i, 0)),
            pl.BlockSpec((BN, 2), lambda i: (i, 0)),
            pl.BlockSpec((1,), lambda i: (0,)),
            pl.BlockSpec((D_OUT, D_OUT), lambda i: (0, 0)),
            pl.BlockSpec((D_IN, D_OUT), lambda i: (0, 0)),
            pl.BlockSpec((D_OUT,), lambda i: (0,)),
        ],
        out_specs=pl.BlockSpec((BN, D_OUT), lambda i: (i, 0)),
    )(tv, H, degt, jnp.reshape(g, (1,)), Wu[:D_OUT], Wu[D_OUT:], bu)
    return out


# speedup vs baseline: 1.1606x; 1.1606x over previous
import functools

import jax
import jax.numpy as jnp
from jax import lax
from jax.experimental import pallas as pl
from jax.experimental.pallas import tpu as pltpu
from jax.experimental.pallas import tpu_sc as plsc

EPS = 1e-12
NC = 2
NS = 16
L = 16
C = 32
NB = 32
EP = 327680
NP = 10240

_MESH = plsc.VectorSubcoreMesh(core_axis_name="c", subcore_axis_name="s",
                               num_cores=NC)


def _make_deg_kernel(deg_pad):
    nch_tile = EP // C // NS
    zrows = deg_pad // NS

    @functools.partial(
        pl.kernel,
        out_type=jax.ShapeDtypeStruct((NC, 1, deg_pad), jnp.float32),
        mesh=_MESH,
        scratch_types=[
            pltpu.VMEM((NB, C), jnp.int32),
            pltpu.VMEM((NB, C), jnp.float32),
            pltpu.VMEM((zrows,), jnp.float32),
            pltpu.VMEM_SHARED((deg_pad,), jnp.float32),
        ],
    )
    def deg_kernel(src2, dst2, w2, out, rows_v, w_v, zero_v, deg_sp):
        c = lax.axis_index("c")
        s = lax.axis_index("s")

        def zb(i, _):
            zero_v[pl.ds(i * L, L)] = jnp.zeros((L,), jnp.float32)
            return 0
        lax.fori_loop(0, zrows // L, zb, 0)
        pltpu.sync_copy(zero_v, deg_sp.at[pl.ds(s * zrows, zrows)])
        plsc.subcore_barrier()

        base = s * nch_tile

        def block(b, _):
            rb = base + b * NB
            pltpu.sync_copy(w2.at[pl.ds(rb, NB)], w_v)

            @pl.when(c == 0)
            def _():
                pltpu.sync_copy(src2.at[pl.ds(rb, NB)], rows_v)

            @pl.when(c != 0)
            def _():
                pltpu.sync_copy(dst2.at[pl.ds(rb, NB)], rows_v)

            def chunk(j, _):
                pltpu.sync_copy(w_v.at[j], deg_sp.at[rows_v.at[j]], add=True)
                return 0
            lax.fori_loop(0, NB, chunk, 0)
            return 0
        lax.fori_loop(0, nch_tile // NB, block, 0)

        plsc.subcore_barrier()
        pltpu.sync_copy(deg_sp.at[pl.ds(s * zrows, zrows)],
                        out.at[c, 0, pl.ds(s * zrows, zrows)])

    return deg_kernel


def _make_spmm_kernel(D):
    nch_tile = EP // C // NS
    rows_out = NP // NS
    BLK = 64
    nblocks = nch_tile // BLK

    @functools.partial(
        pl.kernel,
        out_type=jax.ShapeDtypeStruct((NC, NP, D), jnp.float32),
        mesh=_MESH,
        scratch_types=[
            pltpu.VMEM((BLK, C), jnp.int32),
            pltpu.VMEM((BLK, C), jnp.int32),
            pltpu.VMEM((BLK, C), jnp.float32),
            pltpu.VMEM((C, D), jnp.float32),
            pltpu.VMEM((C, D), jnp.float32),
            pltpu.VMEM((C, D), jnp.float32),
            pltpu.VMEM((C, D), jnp.float32),
            pltpu.SemaphoreType.DMA,
            pltpu.SemaphoreType.DMA,
            pltpu.SemaphoreType.DMA,
            pltpu.SemaphoreType.DMA,
            pltpu.VMEM_SHARED((NP, D), jnp.float32),
        ],
    )
    def spmm_kernel(x, src2, dst2, w2, out,
                    rows_v, cols_v, w_v, g0, g1, s0, s1,
                    gsem0, gsem1, ssem0, ssem1, acc_sp):
        c = lax.axis_index("c")
        s = lax.axis_index("s")
        G = (g0, g1)
        S = (s0, s1)
        GSEM = (gsem0, gsem1)
        SSEM = (ssem0, ssem1)
        dummy = x.at[pl.ds(0, C)]

        def zrow(e, _):
            for k in range(D // L):
                s0[e, pl.ds(k * L, L)] = jnp.zeros((L,), jnp.float32)
            return 0
        lax.fori_loop(0, C, zrow, 0)
        for m in range(rows_out // C):
            pltpu.sync_copy(s0, acc_sp.at[pl.ds(s * rows_out + m * C, C)])
        plsc.subcore_barrier()

        base = s * nch_tile

        def fire_gather(j, b):
            pltpu.async_copy(x.at[cols_v.at[j]], G[b], GSEM[b])

        for blk in range(nblocks):
            rb = base + blk * BLK
            pltpu.sync_copy(w2.at[pl.ds(rb, BLK)], w_v)

            @pl.when(c == 0)
            def _():
                pltpu.sync_copy(src2.at[pl.ds(rb, BLK)], rows_v)
                pltpu.sync_copy(dst2.at[pl.ds(rb, BLK)], cols_v)

            @pl.when(c != 0)
            def _():
                pltpu.sync_copy(dst2.at[pl.ds(rb, BLK)], rows_v)
                pltpu.sync_copy(src2.at[pl.ds(rb, BLK)], cols_v)

            fire_gather(0, 0)
            fire_gather(1, 1)

            def pair(i, _):
                for b in range(2):
                    j = i * 2 + b
                    pltpu.make_async_copy(dummy, G[b], GSEM[b]).wait()
                    @pl.when(j >= 2)
                    def _():
                        pltpu.make_async_copy(dummy, S[b], SSEM[b]).wait()

                    def sgroup(gi, _):
                        wvec = w_v[j, pl.ds(gi * L, L)]
                        for lane in range(L):
                            wv = jnp.full((L,), wvec[lane], jnp.float32)
                            e = gi * L + lane
                            for k in range(D // L):
                                sl = pl.ds(k * L, L)
                                S[b][e, sl] = G[b][e, sl] * wv
                        return 0
                    lax.fori_loop(0, C // L, sgroup, 0)

                    pltpu.async_copy(S[b], acc_sp.at[rows_v.at[j]],
                                     SSEM[b], add=True)

                    @pl.when(j + 2 < BLK)
                    def _():
                        fire_gather(j + 2, b)
                return 0
            lax.fori_loop(0, BLK // 2, pair, 0)

            pltpu.make_async_copy(dummy, S[0], SSEM[0]).wait()
            pltpu.make_async_copy(dummy, S[1], SSEM[1]).wait()

        plsc.subcore_barrier()
        pltpu.sync_copy(acc_sp.at[pl.ds(s * rows_out, rows_out)],
                        out.at[c, pl.ds(s * rows_out, rows_out)])

    return spmm_kernel


def _edge_mlp_body(ea_ref, we1_ref, be1_ref, we2_ref, be2_ref, o_ref):
    eh = jnp.maximum(
        jnp.dot(ea_ref[...], we1_ref[...],
                preferred_element_type=jnp.float32) + be1_ref[...], 0.0)
    logit = jnp.dot(eh, we2_ref[...],
                    preferred_element_type=jnp.float32) + be2_ref[0]
    o_ref[...] = jax.nn.sigmoid(logit)


def _hs_body(h_ref, degt_ref, hs_ref):
    degt = degt_ref[...]
    dis = lax.rsqrt(degt[:, 0] + degt[:, 1] + EPS)
    hs_ref[...] = h_ref[...] * dis[:, None]


def _mid_body(m_ref, degt_ref, msum_ref, xb_ref):
    degt = degt_ref[...]
    deg = degt[:, 0] + degt[:, 1] + EPS
    m = m_ref[0] + m_ref[1]
    msum_ref[...] = m
    xb_ref[...] = m * (1.0 / deg)[:, None]


def _final_body(m_ref, m2_ref, h_ref, degt_ref, g_ref, w1_ref, w2_ref,
                wua_ref, wuh_ref, bu_ref, o_ref):
    gc = jnp.clip(g_ref[0], 0.0, 1.0)
    degt = degt_ref[...]
    dis = lax.rsqrt(degt[:, 0] + degt[:, 1] + EPS)
    m2 = m2_ref[0] + m2_ref[1]
    t = jnp.dot(m_ref[...], w1_ref[...], preferred_element_type=jnp.float32)
    v = jnp.dot(m2, w2_ref[...], preferred_element_type=jnp.float32)
    agg = (gc * t + (1.0 - gc) * v) * dis[:, None]
    acc = jnp.dot(agg, wua_ref[...], preferred_element_type=jnp.float32)
    acc += jnp.dot(h_ref[...], wuh_ref[...],
                   preferred_element_type=jnp.float32)
    o_ref[...] = jnp.maximum(acc + bu_ref[...], 0.0)


def kernel(H, edge_index, edge_attr, W_self, b_self, W1, W2, We1, be1, We2,
           be2, g, Wu, bu):
    N, D_IN = H.shape
    E, D_EDGE = edge_attr.shape
    D_OUT = Wu.shape[1]
    H_EDGE = We1.shape[1]
    deg_pad = NP

    BE = 8000
    edge_w = pl.pallas_call(
        _edge_mlp_body,
        out_shape=jax.ShapeDtypeStruct((E, 1), jnp.float32),
        grid=(E // BE,),
        in_specs=[
            pl.BlockSpec((BE, D_EDGE), lambda i: (i, 0)),
            pl.BlockSpec((D_EDGE, H_EDGE), lambda i: (0, 0)),
            pl.BlockSpec((H_EDGE,), lambda i: (0,)),
            pl.BlockSpec((H_EDGE, 1), lambda i: (0, 0)),
            pl.BlockSpec((1,), lambda i: (0,)),
        ],
        out_specs=pl.BlockSpec((BE, 1), lambda i: (i, 0)),
    )(edge_attr, We1, be1, We2, be2)

    pad = EP - E
    src2 = jnp.concatenate([edge_index[0],
                            jnp.zeros((pad,), jnp.int32)]).reshape(EP // C, C)
    dst2 = jnp.concatenate([edge_index[1],
                            jnp.zeros((pad,), jnp.int32)]).reshape(EP // C, C)
    w2 = jnp.concatenate([edge_w[:, 0],
                          jnp.zeros((pad,), jnp.float32)]).reshape(EP // C, C)

    degp = _make_deg_kernel(deg_pad)(src2, dst2, w2)
    degt = degp[:, 0, :N].T

    BN = 1000
    hs = pl.pallas_call(
        _hs_body,
        out_shape=jax.ShapeDtypeStruct((N, D_IN), jnp.float32),
        grid=(N // BN,),
        in_specs=[
            pl.BlockSpec((BN, D_IN), lambda i: (i, 0)),
            pl.BlockSpec((BN, 2), lambda i: (i, 0)),
        ],
        out_specs=pl.BlockSpec((BN, D_IN), lambda i: (i, 0)),
    )(H, degt)

    spmm = _make_spmm_kernel(D_IN)

    mp = spmm(hs, src2, dst2, w2)

    msum, xb = pl.pallas_call(
        _mid_body,
        out_shape=[jax.ShapeDtypeStruct((N, D_IN), jnp.float32),
                   jax.ShapeDtypeStruct((N, D_IN), jnp.float32)],
        grid=(N // BN,),
        in_specs=[
            pl.BlockSpec((NC, BN, D_IN), lambda i: (0, i, 0)),
            pl.BlockSpec((BN, 2), lambda i: (i, 0)),
        ],
        out_specs=[pl.BlockSpec((BN, D_IN), lambda i: (i, 0)),
                   pl.BlockSpec((BN, D_IN), lambda i: (i, 0))],
    )(mp, degt)

    m2p = spmm(xb, src2, dst2, w2)

    out = pl.pallas_call(
        _final_body,
        out_shape=jax.ShapeDtypeStruct((N, D_OUT), jnp.float32),
        grid=(N // BN,),
        in_specs=[
            pl.BlockSpec((BN, D_IN), lambda i: (i, 0)),
            pl.BlockSpec((NC, BN, D_IN), lambda i: (0, i, 0)),
            pl.BlockSpec((BN, D_IN), lambda i: (i, 0)),
            pl.BlockSpec((BN, 2), lambda i: (i, 0)),
            pl.BlockSpec((1,), lambda i: (0,)),
            pl.BlockSpec((D_IN, D_OUT), lambda i: (0, 0)),
            pl.BlockSpec((D_IN, D_OUT), lambda i: (0, 0)),
            pl.BlockSpec((D_OUT, D_OUT), lambda i: (0, 0)),
            pl.BlockSpec((D_IN, D_OUT), lambda i: (0, 0)),
            pl.BlockSpec((D_OUT,), lambda i: (0,)),
        ],
        out_specs=pl.BlockSpec((BN, D_OUT), lambda i: (i, 0)),
    )(msum, m2p, H, degt, jnp.reshape(g, (1,)), W1, W2,
      Wu[:D_OUT], Wu[D_OUT:], bu)
    return out


# speedup vs baseline: 1.1923x; 1.0273x over previous
import functools

import jax
import jax.numpy as jnp
from jax import lax
from jax.experimental import pallas as pl
from jax.experimental.pallas import tpu as pltpu
from jax.experimental.pallas import tpu_sc as plsc

EPS = 1e-12
NC = 2
NS = 16
L = 16
C = 32
NB = 32
EP = 327680
NP = 10240

_MESH = plsc.VectorSubcoreMesh(core_axis_name="c", subcore_axis_name="s",
                               num_cores=NC)


CD = 128


def _make_deg_kernel(deg_pad):
    nch_tile = EP // CD // NS
    zrows = deg_pad // NS

    @functools.partial(
        pl.kernel,
        out_type=jax.ShapeDtypeStruct((NC, 1, deg_pad), jnp.float32),
        mesh=_MESH,
        scratch_types=[
            pltpu.VMEM((NB, CD), jnp.int32),
            pltpu.VMEM((NB, CD), jnp.float32),
            pltpu.VMEM((zrows,), jnp.float32),
            pltpu.VMEM_SHARED((deg_pad,), jnp.float32),
        ],
    )
    def deg_kernel(src2, dst2, w2, out, rows_v, w_v, zero_v, deg_sp):
        c = lax.axis_index("c")
        s = lax.axis_index("s")

        def zb(i, _):
            zero_v[pl.ds(i * L, L)] = jnp.zeros((L,), jnp.float32)
            return 0
        lax.fori_loop(0, zrows // L, zb, 0)
        pltpu.sync_copy(zero_v, deg_sp.at[pl.ds(s * zrows, zrows)])
        plsc.subcore_barrier()

        base = s * nch_tile

        def block(b, _):
            rb = base + b * NB
            pltpu.sync_copy(w2.at[pl.ds(rb, NB)], w_v)

            @pl.when(c == 0)
            def _():
                pltpu.sync_copy(src2.at[pl.ds(rb, NB)], rows_v)

            @pl.when(c != 0)
            def _():
                pltpu.sync_copy(dst2.at[pl.ds(rb, NB)], rows_v)

            def chunk(j, _):
                pltpu.sync_copy(w_v.at[j], deg_sp.at[rows_v.at[j]], add=True)
                return 0
            lax.fori_loop(0, NB, chunk, 0)
            return 0
        lax.fori_loop(0, nch_tile // NB, block, 0)

        plsc.subcore_barrier()
        pltpu.sync_copy(deg_sp.at[pl.ds(s * zrows, zrows)],
                        out.at[c, 0, pl.ds(s * zrows, zrows)])

    return deg_kernel


def _make_spmm_kernel(D):
    nch_tile = EP // C // NS
    rows_out = NP // NS
    BLK = 64
    nblocks = nch_tile // BLK

    @functools.partial(
        pl.kernel,
        out_type=jax.ShapeDtypeStruct((NC, NP, D), jnp.float32),
        mesh=_MESH,
        scratch_types=[
            pltpu.VMEM((BLK, C), jnp.int32),
            pltpu.VMEM((BLK, C), jnp.int32),
            pltpu.VMEM((BLK, C), jnp.float32),
            pltpu.VMEM((C, D), jnp.float32),
            pltpu.VMEM((C, D), jnp.float32),
            pltpu.VMEM((C, D), jnp.float32),
            pltpu.VMEM((C, D), jnp.float32),
            pltpu.SemaphoreType.DMA,
            pltpu.SemaphoreType.DMA,
            pltpu.SemaphoreType.DMA,
            pltpu.SemaphoreType.DMA,
            pltpu.VMEM_SHARED((NP, D), jnp.float32),
        ],
    )
    def spmm_kernel(x, src2, dst2, w2, out,
                    rows_v, cols_v, w_v, g0, g1, s0, s1,
                    gsem0, gsem1, ssem0, ssem1, acc_sp):
        c = lax.axis_index("c")
        s = lax.axis_index("s")
        G = (g0, g1)
        S = (s0, s1)
        GSEM = (gsem0, gsem1)
        SSEM = (ssem0, ssem1)
        dummy = x.at[pl.ds(0, C)]

        def zrow(e, _):
            for k in range(D // L):
                s0[e, pl.ds(k * L, L)] = jnp.zeros((L,), jnp.float32)
            return 0
        lax.fori_loop(0, C, zrow, 0)
        for m in range(rows_out // C):
            pltpu.sync_copy(s0, acc_sp.at[pl.ds(s * rows_out + m * C, C)])
        plsc.subcore_barrier()

        base = s * nch_tile

        def fire_gather(j, b):
            pltpu.async_copy(x.at[cols_v.at[j]], G[b], GSEM[b])

        for blk in range(nblocks):
            rb = base + blk * BLK
            pltpu.sync_copy(w2.at[pl.ds(rb, BLK)], w_v)

            @pl.when(c == 0)
            def _():
                pltpu.sync_copy(src2.at[pl.ds(rb, BLK)], rows_v)
                pltpu.sync_copy(dst2.at[pl.ds(rb, BLK)], cols_v)

            @pl.when(c != 0)
            def _():
                pltpu.sync_copy(dst2.at[pl.ds(rb, BLK)], rows_v)
                pltpu.sync_copy(src2.at[pl.ds(rb, BLK)], cols_v)

            fire_gather(0, 0)
            fire_gather(1, 1)

            def pair(i, _):
                for b in range(2):
                    j = i * 2 + b
                    pltpu.make_async_copy(dummy, G[b], GSEM[b]).wait()
                    @pl.when(j >= 2)
                    def _():
                        pltpu.make_async_copy(dummy, S[b], SSEM[b]).wait()

                    def sgroup(gi, _):
                        wvec = w_v[j, pl.ds(gi * L, L)]
                        for lane in range(L):
                            wv = jnp.full((L,), wvec[lane], jnp.float32)
                            e = gi * L + lane
                            for k in range(D // L):
                                sl = pl.ds(k * L, L)
                                S[b][e, sl] = G[b][e, sl] * wv
                        return 0
                    lax.fori_loop(0, C // L, sgroup, 0)

                    pltpu.async_copy(S[b], acc_sp.at[rows_v.at[j]],
                                     SSEM[b], add=True)

                    @pl.when(j + 2 < BLK)
                    def _():
                        fire_gather(j + 2, b)
                return 0
            lax.fori_loop(0, BLK // 2, pair, 0)

            pltpu.make_async_copy(dummy, S[0], SSEM[0]).wait()
            pltpu.make_async_copy(dummy, S[1], SSEM[1]).wait()

        plsc.subcore_barrier()
        pltpu.sync_copy(acc_sp.at[pl.ds(s * rows_out, rows_out)],
                        out.at[c, pl.ds(s * rows_out, rows_out)])

    return spmm_kernel


def _edge_mlp_body(ea_ref, we1_ref, be1_ref, we2_ref, be2_ref, o_ref):
    eh = jnp.maximum(
        jnp.dot(ea_ref[...], we1_ref[...],
                preferred_element_type=jnp.float32) + be1_ref[...], 0.0)
    logit = jnp.dot(eh, we2_ref[...],
                    preferred_element_type=jnp.float32) + be2_ref[0]
    o_ref[...] = jax.nn.sigmoid(logit)


def _hs_body(h_ref, degt_ref, hs_ref):
    degt = degt_ref[...]
    dis = lax.rsqrt(degt[:, 0] + degt[:, 1] + EPS)
    hs_ref[...] = h_ref[...] * dis[:, None]


def _mid_body(m_ref, degt_ref, msum_ref, xb_ref):
    degt = degt_ref[...]
    deg = degt[:, 0] + degt[:, 1] + EPS
    m = m_ref[0] + m_ref[1]
    msum_ref[...] = m
    xb_ref[...] = m * (1.0 / deg)[:, None]


def _final_body(m_ref, m2_ref, h_ref, degt_ref, g_ref, w1_ref, w2_ref,
                wua_ref, wuh_ref, bu_ref, o_ref):
    gc = jnp.clip(g_ref[0], 0.0, 1.0)
    degt = degt_ref[...]
    dis = lax.rsqrt(degt[:, 0] + degt[:, 1] + EPS)
    m2 = m2_ref[0] + m2_ref[1]
    t = jnp.dot(m_ref[...], w1_ref[...], preferred_element_type=jnp.float32)
    v = jnp.dot(m2, w2_ref[...], preferred_element_type=jnp.float32)
    agg = (gc * t + (1.0 - gc) * v) * dis[:, None]
    acc = jnp.dot(agg, wua_ref[...], preferred_element_type=jnp.float32)
    acc += jnp.dot(h_ref[...], wuh_ref[...],
                   preferred_element_type=jnp.float32)
    o_ref[...] = jnp.maximum(acc + bu_ref[...], 0.0)


def kernel(H, edge_index, edge_attr, W_self, b_self, W1, W2, We1, be1, We2,
           be2, g, Wu, bu):
    N, D_IN = H.shape
    E, D_EDGE = edge_attr.shape
    D_OUT = Wu.shape[1]
    H_EDGE = We1.shape[1]
    deg_pad = NP

    BE = 8000
    edge_w = pl.pallas_call(
        _edge_mlp_body,
        out_shape=jax.ShapeDtypeStruct((E, 1), jnp.float32),
        grid=(E // BE,),
        in_specs=[
            pl.BlockSpec((BE, D_EDGE), lambda i: (i, 0)),
            pl.BlockSpec((D_EDGE, H_EDGE), lambda i: (0, 0)),
            pl.BlockSpec((H_EDGE,), lambda i: (0,)),
            pl.BlockSpec((H_EDGE, 1), lambda i: (0, 0)),
            pl.BlockSpec((1,), lambda i: (0,)),
        ],
        out_specs=pl.BlockSpec((BE, 1), lambda i: (i, 0)),
    )(edge_attr, We1, be1, We2, be2)

    pad = EP - E
    src2 = jnp.concatenate([edge_index[0],
                            jnp.zeros((pad,), jnp.int32)]).reshape(EP // C, C)
    dst2 = jnp.concatenate([edge_index[1],
                            jnp.zeros((pad,), jnp.int32)]).reshape(EP // C, C)
    w2 = jnp.concatenate([edge_w[:, 0],
                          jnp.zeros((pad,), jnp.float32)]).reshape(EP // C, C)

    srcd = src2.reshape(EP // CD, CD)
    dstd = dst2.reshape(EP // CD, CD)
    wd = w2.reshape(EP // CD, CD)
    degp = _make_deg_kernel(deg_pad)(srcd, dstd, wd)
    degt = degp[:, 0, :N].T

    BN = 1000
    hs = pl.pallas_call(
        _hs_body,
        out_shape=jax.ShapeDtypeStruct((N, D_IN), jnp.float32),
        grid=(N // BN,),
        in_specs=[
            pl.BlockSpec((BN, D_IN), lambda i: (i, 0)),
            pl.BlockSpec((BN, 2), lambda i: (i, 0)),
        ],
        out_specs=pl.BlockSpec((BN, D_IN), lambda i: (i, 0)),
    )(H, degt)

    spmm = _make_spmm_kernel(D_IN)

    mp = spmm(hs, src2, dst2, w2)

    msum, xb = pl.pallas_call(
        _mid_body,
        out_shape=[jax.ShapeDtypeStruct((N, D_IN), jnp.float32),
                   jax.ShapeDtypeStruct((N, D_IN), jnp.float32)],
        grid=(N // BN,),
        in_specs=[
            pl.BlockSpec((NC, BN, D_IN), lambda i: (0, i, 0)),
            pl.BlockSpec((BN, 2), lambda i: (i, 0)),
        ],
        out_specs=[pl.BlockSpec((BN, D_IN), lambda i: (i, 0)),
                   pl.BlockSpec((BN, D_IN), lambda i: (i, 0))],
    )(mp, degt)

    m2p = spmm(xb, src2, dst2, w2)

    out = pl.pallas_call(
        _final_body,
        out_shape=jax.ShapeDtypeStruct((N, D_OUT), jnp.float32),
        grid=(N // BN,),
        in_specs=[
            pl.BlockSpec((BN, D_IN), lambda i: (i, 0)),
            pl.BlockSpec((NC, BN, D_IN), lambda i: (0, i, 0)),
            pl.BlockSpec((BN, D_IN), lambda i: (i, 0)),
            pl.BlockSpec((BN, 2), lambda i: (i, 0)),
            pl.BlockSpec((1,), lambda i: (0,)),
            pl.BlockSpec((D_IN, D_OUT), lambda i: (0, 0)),
            pl.BlockSpec((D_IN, D_OUT), lambda i: (0, 0)),
            pl.BlockSpec((D_OUT, D_OUT), lambda i: (0, 0)),
            pl.BlockSpec((D_IN, D_OUT), lambda i: (0, 0)),
            pl.BlockSpec((D_OUT,), lambda i: (0,)),
        ],
        out_specs=pl.BlockSpec((BN, D_OUT), lambda i: (i, 0)),
    )(msum, m2p, H, degt, jnp.reshape(g, (1,)), W1, W2,
      Wu[:D_OUT], Wu[D_OUT:], bu)
    return out


# speedup vs baseline: 1.2171x; 1.0208x over previous
import functools

import jax
import jax.numpy as jnp
from jax import lax
from jax.experimental import pallas as pl
from jax.experimental.pallas import tpu as pltpu
from jax.experimental.pallas import tpu_sc as plsc

EPS = 1e-12
NC = 2
NS = 16
L = 16
C = 32
NB = 32
EP = 327680
NP = 10240

_MESH = plsc.VectorSubcoreMesh(core_axis_name="c", subcore_axis_name="s",
                               num_cores=NC)


CD = 128


def _make_deg_kernel(deg_pad):
    nch_tile = EP // CD // NS
    zrows = deg_pad // NS

    @functools.partial(
        pl.kernel,
        out_type=jax.ShapeDtypeStruct((NC, 1, deg_pad), jnp.float32),
        mesh=_MESH,
        scratch_types=[
            pltpu.VMEM((NB, CD), jnp.int32),
            pltpu.VMEM((NB, CD), jnp.float32),
            pltpu.VMEM((zrows,), jnp.float32),
            pltpu.VMEM_SHARED((deg_pad,), jnp.float32),
        ],
    )
    def deg_kernel(src2, dst2, w2, out, rows_v, w_v, zero_v, deg_sp):
        c = lax.axis_index("c")
        s = lax.axis_index("s")

        def zb(i, _):
            zero_v[pl.ds(i * L, L)] = jnp.zeros((L,), jnp.float32)
            return 0
        lax.fori_loop(0, zrows // L, zb, 0)
        pltpu.sync_copy(zero_v, deg_sp.at[pl.ds(s * zrows, zrows)])
        plsc.subcore_barrier()

        base = s * nch_tile

        def block(b, _):
            rb = base + b * NB
            pltpu.sync_copy(w2.at[pl.ds(rb, NB)], w_v)

            @pl.when(c == 0)
            def _():
                pltpu.sync_copy(src2.at[pl.ds(rb, NB)], rows_v)

            @pl.when(c != 0)
            def _():
                pltpu.sync_copy(dst2.at[pl.ds(rb, NB)], rows_v)

            def chunk(j, _):
                pltpu.sync_copy(w_v.at[j], deg_sp.at[rows_v.at[j]], add=True)
                return 0
            lax.fori_loop(0, NB, chunk, 0)
            return 0
        lax.fori_loop(0, nch_tile // NB, block, 0)

        plsc.subcore_barrier()
        pltpu.sync_copy(deg_sp.at[pl.ds(s * zrows, zrows)],
                        out.at[c, 0, pl.ds(s * zrows, zrows)])

    return deg_kernel


def _make_spmm_kernel(D):
    nch_tile = EP // C // NS
    rows_out = NP // NS
    BLK = 32
    nblocks = nch_tile // BLK

    @functools.partial(
        pl.kernel,
        out_type=jax.ShapeDtypeStruct((NC, NP, D), jnp.float32),
        mesh=_MESH,
        scratch_types=[
            pltpu.VMEM((BLK, C), jnp.int32),
            pltpu.VMEM((BLK, C), jnp.int32),
            pltpu.VMEM((BLK, C), jnp.float32),
            pltpu.VMEM((C, D), jnp.float32),
            pltpu.VMEM((C, D), jnp.float32),
            pltpu.VMEM((C, D), jnp.float32),
            pltpu.VMEM((C, D), jnp.float32),
            pltpu.VMEM((C, D), jnp.float32),
            pltpu.SemaphoreType.DMA,
            pltpu.SemaphoreType.DMA,
            pltpu.SemaphoreType.DMA,
            pltpu.SemaphoreType.DMA,
            pltpu.SemaphoreType.DMA,
            pltpu.VMEM_SHARED((NP, D), jnp.float32),
        ],
    )
    def spmm_kernel(x, src2, dst2, w2, out,
                    rows_v, cols_v, w_v, g0, g1, g2, g3, s0,
                    gsem0, gsem1, gsem2, gsem3, ssem0, acc_sp):
        c = lax.axis_index("c")
        s = lax.axis_index("s")
        G = (g0, g1, g2, g3)
        GSEM = (gsem0, gsem1, gsem2, gsem3)
        dummy = x.at[pl.ds(0, C)]

        def zrow(e, _):
            for k in range(D // L):
                s0[e, pl.ds(k * L, L)] = jnp.zeros((L,), jnp.float32)
            return 0
        lax.fori_loop(0, C, zrow, 0)
        for m in range(rows_out // C):
            pltpu.sync_copy(s0, acc_sp.at[pl.ds(s * rows_out + m * C, C)])
        plsc.subcore_barrier()

        base = s * nch_tile

        def fire_gather(j, b):
            pltpu.async_copy(x.at[cols_v.at[j]], G[b], GSEM[b])

        def blockfn(blk, _):
            rb = pl.multiple_of(base + blk * BLK, 8)
            pltpu.sync_copy(w2.at[pl.ds(rb, BLK)], w_v)

            @pl.when(c == 0)
            def _():
                pltpu.sync_copy(src2.at[pl.ds(rb, BLK)], rows_v)
                pltpu.sync_copy(dst2.at[pl.ds(rb, BLK)], cols_v)

            @pl.when(c != 0)
            def _():
                pltpu.sync_copy(dst2.at[pl.ds(rb, BLK)], rows_v)
                pltpu.sync_copy(src2.at[pl.ds(rb, BLK)], cols_v)

            for b in range(4):
                fire_gather(b, b)

            def quad(i, _):
                for b in range(4):
                    j = i * 4 + b
                    pltpu.make_async_copy(dummy, G[b], GSEM[b]).wait()
                    @pl.when(j >= 1)
                    def _():
                        pltpu.make_async_copy(dummy, s0, ssem0).wait()

                    def sgroup(gi, _):
                        wvec = w_v[j, pl.ds(gi * L, L)]
                        for lane in range(L):
                            wv = jnp.full((L,), wvec[lane], jnp.float32)
                            e = gi * L + lane
                            for k in range(D // L):
                                sl = pl.ds(k * L, L)
                                s0[e, sl] = G[b][e, sl] * wv
                        return 0
                    lax.fori_loop(0, C // L, sgroup, 0)

                    pltpu.async_copy(s0, acc_sp.at[rows_v.at[j]],
                                     ssem0, add=True)

                    @pl.when(j + 4 < BLK)
                    def _():
                        fire_gather(j + 4, b)
                return 0
            lax.fori_loop(0, BLK // 4, quad, 0)

            pltpu.make_async_copy(dummy, s0, ssem0).wait()
            return 0
        lax.fori_loop(0, nblocks, blockfn, 0)

        plsc.subcore_barrier()
        pltpu.sync_copy(acc_sp.at[pl.ds(s * rows_out, rows_out)],
                        out.at[c, pl.ds(s * rows_out, rows_out)])

    return spmm_kernel


def _edge_mlp_body(ea_ref, we1_ref, be1_ref, we2_ref, be2_ref, o_ref):
    eh = jnp.maximum(
        jnp.dot(ea_ref[...], we1_ref[...],
                preferred_element_type=jnp.float32) + be1_ref[...], 0.0)
    logit = jnp.dot(eh, we2_ref[...],
                    preferred_element_type=jnp.float32) + be2_ref[0]
    o_ref[...] = jax.nn.sigmoid(logit)


def _hs_body(h_ref, degt_ref, hs_ref):
    degt = degt_ref[...]
    dis = lax.rsqrt(degt[:, 0] + degt[:, 1] + EPS)
    hs_ref[...] = h_ref[...] * dis[:, None]


def _mid_body(m_ref, degt_ref, msum_ref, xb_ref):
    degt = degt_ref[...]
    deg = degt[:, 0] + degt[:, 1] + EPS
    m = m_ref[0] + m_ref[1]
    msum_ref[...] = m
    xb_ref[...] = m * (1.0 / deg)[:, None]


def _final_body(m_ref, m2_ref, h_ref, degt_ref, g_ref, w1_ref, w2_ref,
                wua_ref, wuh_ref, bu_ref, o_ref):
    gc = jnp.clip(g_ref[0], 0.0, 1.0)
    degt = degt_ref[...]
    dis = lax.rsqrt(degt[:, 0] + degt[:, 1] + EPS)
    m2 = m2_ref[0] + m2_ref[1]
    t = jnp.dot(m_ref[...], w1_ref[...], preferred_element_type=jnp.float32)
    v = jnp.dot(m2, w2_ref[...], preferred_element_type=jnp.float32)
    agg = (gc * t + (1.0 - gc) * v) * dis[:, None]
    acc = jnp.dot(agg, wua_ref[...], preferred_element_type=jnp.float32)
    acc += jnp.dot(h_ref[...], wuh_ref[...],
                   preferred_element_type=jnp.float32)
    o_ref[...] = jnp.maximum(acc + bu_ref[...], 0.0)


def kernel(H, edge_index, edge_attr, W_self, b_self, W1, W2, We1, be1, We2,
           be2, g, Wu, bu):
    N, D_IN = H.shape
    E, D_EDGE = edge_attr.shape
    D_OUT = Wu.shape[1]
    H_EDGE = We1.shape[1]
    deg_pad = NP

    BE = 8000
    edge_w = pl.pallas_call(
        _edge_mlp_body,
        out_shape=jax.ShapeDtypeStruct((E, 1), jnp.float32),
        grid=(E // BE,),
        in_specs=[
            pl.BlockSpec((BE, D_EDGE), lambda i: (i, 0)),
            pl.BlockSpec((D_EDGE, H_EDGE), lambda i: (0, 0)),
            pl.BlockSpec((H_EDGE,), lambda i: (0,)),
            pl.BlockSpec((H_EDGE, 1), lambda i: (0, 0)),
            pl.BlockSpec((1,), lambda i: (0,)),
        ],
        out_specs=pl.BlockSpec((BE, 1), lambda i: (i, 0)),
    )(edge_attr, We1, be1, We2, be2)

    pad = EP - E
    src2 = jnp.concatenate([edge_index[0],
                            jnp.zeros((pad,), jnp.int32)]).reshape(EP // C, C)
    dst2 = jnp.concatenate([edge_index[1],
                            jnp.zeros((pad,), jnp.int32)]).reshape(EP // C, C)
    w2 = jnp.concatenate([edge_w[:, 0],
                          jnp.zeros((pad,), jnp.float32)]).reshape(EP // C, C)

    srcd = src2.reshape(EP // CD, CD)
    dstd = dst2.reshape(EP // CD, CD)
    wd = w2.reshape(EP // CD, CD)
    degp = _make_deg_kernel(deg_pad)(srcd, dstd, wd)
    degt = degp[:, 0, :N].T

    BN = 1000
    hs = pl.pallas_call(
        _hs_body,
        out_shape=jax.ShapeDtypeStruct((N, D_IN), jnp.float32),
        grid=(N // BN,),
        in_specs=[
            pl.BlockSpec((BN, D_IN), lambda i: (i, 0)),
            pl.BlockSpec((BN, 2), lambda i: (i, 0)),
        ],
        out_specs=pl.BlockSpec((BN, D_IN), lambda i: (i, 0)),
    )(H, degt)

    spmm = _make_spmm_kernel(D_IN)

    mp = spmm(hs, src2, dst2, w2)

    msum, xb = pl.pallas_call(
        _mid_body,
        out_shape=[jax.ShapeDtypeStruct((N, D_IN), jnp.float32),
                   jax.ShapeDtypeStruct((N, D_IN), jnp.float32)],
        grid=(N // BN,),
        in_specs=[
            pl.BlockSpec((NC, BN, D_IN), lambda i: (0, i, 0)),
            pl.BlockSpec((BN, 2), lambda i: (i, 0)),
        ],
        out_specs=[pl.BlockSpec((BN, D_IN), lambda i: (i, 0)),
                   pl.BlockSpec((BN, D_IN), lambda i: (i, 0))],
    )(mp, degt)

    m2p = spmm(xb, src2, dst2, w2)

    out = pl.pallas_call(
        _final_body,
        out_shape=jax.ShapeDtypeStruct((N, D_OUT), jnp.float32),
        grid=(N // BN,),
        in_specs=[
            pl.BlockSpec((BN, D_IN), lambda i: (i, 0)),
            pl.BlockSpec((NC, BN, D_IN), lambda i: (0, i, 0)),
            pl.BlockSpec((BN, D_IN), lambda i: (i, 0)),
            pl.BlockSpec((BN, 2), lambda i: (i, 0)),
            pl.BlockSpec((1,), lambda i: (0,)),
            pl.BlockSpec((D_IN, D_OUT), lambda i: (0, 0)),
            pl.BlockSpec((D_IN, D_OUT), lambda i: (0, 0)),
            pl.BlockSpec((D_OUT, D_OUT), lambda i: (0, 0)),
            pl.BlockSpec((D_IN, D_OUT), lambda i: (0, 0)),
            pl.BlockSpec((D_OUT,), lambda i: (0,)),
        ],
        out_specs=pl.BlockSpec((BN, D_OUT), lambda i: (i, 0)),
    )(msum, m2p, H, degt, jnp.reshape(g, (1,)), W1, W2,
      Wu[:D_OUT], Wu[D_OUT:], bu)
    return out


# speedup vs baseline: 1.2218x; 1.0039x over previous
import functools

import jax
import jax.numpy as jnp
from jax import lax
from jax.experimental import pallas as pl
from jax.experimental.pallas import tpu as pltpu
from jax.experimental.pallas import tpu_sc as plsc

EPS = 1e-12
NC = 2
NS = 16
L = 16
C = 32
NB = 32
EP = 327680
NP = 10240

_MESH = plsc.VectorSubcoreMesh(core_axis_name="c", subcore_axis_name="s",
                               num_cores=NC)


CD = 128


def _make_deg_kernel(deg_pad):
    nch_tile = EP // CD // NS
    zrows = deg_pad // NS

    @functools.partial(
        pl.kernel,
        out_type=jax.ShapeDtypeStruct((NC, 1, deg_pad), jnp.float32),
        mesh=_MESH,
        scratch_types=[
            pltpu.VMEM((NB, CD), jnp.int32),
            pltpu.VMEM((NB, CD), jnp.float32),
            pltpu.VMEM((zrows,), jnp.float32),
            pltpu.VMEM_SHARED((deg_pad,), jnp.float32),
        ],
    )
    def deg_kernel(src2, dst2, w2, out, rows_v, w_v, zero_v, deg_sp):
        c = lax.axis_index("c")
        s = lax.axis_index("s")

        def zb(i, _):
            zero_v[pl.ds(i * L, L)] = jnp.zeros((L,), jnp.float32)
            return 0
        lax.fori_loop(0, zrows // L, zb, 0)
        pltpu.sync_copy(zero_v, deg_sp.at[pl.ds(s * zrows, zrows)])
        plsc.subcore_barrier()

        base = s * nch_tile

        def block(b, _):
            rb = base + b * NB
            pltpu.sync_copy(w2.at[pl.ds(rb, NB)], w_v)

            @pl.when(c == 0)
            def _():
                pltpu.sync_copy(src2.at[pl.ds(rb, NB)], rows_v)

            @pl.when(c != 0)
            def _():
                pltpu.sync_copy(dst2.at[pl.ds(rb, NB)], rows_v)

            def chunk(j, _):
                pltpu.sync_copy(w_v.at[j], deg_sp.at[rows_v.at[j]], add=True)
                return 0
            lax.fori_loop(0, NB, chunk, 0)
            return 0
        lax.fori_loop(0, nch_tile // NB, block, 0)

        plsc.subcore_barrier()
        pltpu.sync_copy(deg_sp.at[pl.ds(s * zrows, zrows)],
                        out.at[c, 0, pl.ds(s * zrows, zrows)])

    return deg_kernel


def _make_spmm_kernel(D):
    nch_tile = EP // C // NS
    rows_out = NP // NS
    BLK = 32
    nblocks = nch_tile // BLK

    @functools.partial(
        pl.kernel,
        out_type=jax.ShapeDtypeStruct((NC, NP, D), jnp.float32),
        mesh=_MESH,
        scratch_types=[
            pltpu.VMEM((BLK, C), jnp.int32),
            pltpu.VMEM((BLK, C), jnp.int32),
            pltpu.VMEM((BLK, C), jnp.float32),
            pltpu.VMEM((C, D), jnp.float32),
            pltpu.VMEM((C, D), jnp.float32),
            pltpu.VMEM((C, D), jnp.float32),
            pltpu.VMEM((C, D), jnp.float32),
            pltpu.VMEM((C, D), jnp.float32),
            pltpu.SemaphoreType.DMA,
            pltpu.SemaphoreType.DMA,
            pltpu.SemaphoreType.DMA,
            pltpu.SemaphoreType.DMA,
            pltpu.SemaphoreType.DMA,
            pltpu.VMEM_SHARED((NP, D), jnp.float32),
        ],
    )
    def spmm_kernel(x, src2, dst2, w2, out,
                    rows_v, cols_v, w_v, g0, g1, g2, g3, s0,
                    gsem0, gsem1, gsem2, gsem3, ssem0, acc_sp):
        c = lax.axis_index("c")
        s = lax.axis_index("s")
        G = (g0, g1, g2, g3)
        GSEM = (gsem0, gsem1, gsem2, gsem3)
        dummy = x.at[pl.ds(0, C)]

        def zrow(e, _):
            for k in range(D // L):
                s0[e, pl.ds(k * L, L)] = jnp.zeros((L,), jnp.float32)
            return 0
        lax.fori_loop(0, C, zrow, 0)
        for m in range(rows_out // C):
            pltpu.sync_copy(s0, acc_sp.at[pl.ds(s * rows_out + m * C, C)])
        plsc.subcore_barrier()

        base = s * nch_tile

        def fire_gather(j, b):
            pltpu.async_copy(x.at[cols_v.at[j]], G[b], GSEM[b])

        def blockfn(blk, _):
            rb = pl.multiple_of(base + blk * BLK, 8)
            pltpu.sync_copy(w2.at[pl.ds(rb, BLK)], w_v)

            @pl.when(c == 0)
            def _():
                pltpu.sync_copy(src2.at[pl.ds(rb, BLK)], rows_v)
                pltpu.sync_copy(dst2.at[pl.ds(rb, BLK)], cols_v)

            @pl.when(c != 0)
            def _():
                pltpu.sync_copy(dst2.at[pl.ds(rb, BLK)], rows_v)
                pltpu.sync_copy(src2.at[pl.ds(rb, BLK)], cols_v)

            for b in range(4):
                fire_gather(b, b)

            def quad(i, _):
                for b in range(4):
                    j = i * 4 + b
                    pltpu.make_async_copy(dummy, G[b], GSEM[b]).wait()
                    @pl.when(j >= 1)
                    def _():
                        pltpu.make_async_copy(dummy, s0, ssem0).wait()

                    def sgroup(gi, _):
                        wvec = w_v[j, pl.ds(gi * L, L)]
                        for lane in range(L):
                            wv = jnp.full((L,), wvec[lane], jnp.float32)
                            e = gi * L + lane
                            for k in range(D // L):
                                sl = pl.ds(k * L, L)
                                s0[e, sl] = G[b][e, sl] * wv
                        return 0
                    lax.fori_loop(0, C // L, sgroup, 0)

                    pltpu.async_copy(s0, acc_sp.at[rows_v.at[j]],
                                     ssem0, add=True)

                    @pl.when(j + 4 < BLK)
                    def _():
                        fire_gather(j + 4, b)
                return 0
            lax.fori_loop(0, BLK // 4, quad, 0)

            pltpu.make_async_copy(dummy, s0, ssem0).wait()
            return 0
        lax.fori_loop(0, nblocks, blockfn, 0)

        plsc.subcore_barrier()
        pltpu.sync_copy(acc_sp.at[pl.ds(s * rows_out, rows_out)],
                        out.at[c, pl.ds(s * rows_out, rows_out)])

    return spmm_kernel


def _edge_mlp_body(ea_ref, we1_ref, be1_ref, we2_ref, be2_ref, o_ref):
    eh = jnp.maximum(
        jnp.dot(ea_ref[...], we1_ref[...],
                preferred_element_type=jnp.float32) + be1_ref[...], 0.0)
    logit = jnp.dot(eh, we2_ref[...],
                    preferred_element_type=jnp.float32) + be2_ref[0]
    o_ref[...] = jax.nn.sigmoid(logit)


def _hs_body(h_ref, degt_ref, hs_ref):
    degt = degt_ref[...]
    dis = lax.rsqrt(degt[:, 0] + degt[:, 1] + EPS)
    hs_ref[...] = h_ref[...] * dis[:, None]


def _mid_body(m_ref, degt_ref, xb_ref):
    degt = degt_ref[...]
    deg = degt[:, 0] + degt[:, 1] + EPS
    m = m_ref[0] + m_ref[1]
    xb_ref[...] = m * (1.0 / deg)[:, None]


def _final_body(m_ref, m2_ref, h_ref, degt_ref, g_ref, w1_ref, w2_ref,
                wua_ref, wuh_ref, bu_ref, o_ref):
    gc = jnp.clip(g_ref[0], 0.0, 1.0)
    degt = degt_ref[...]
    dis = lax.rsqrt(degt[:, 0] + degt[:, 1] + EPS)
    m2 = m2_ref[0] + m2_ref[1]
    m = m_ref[0] + m_ref[1]
    t = jnp.dot(m, w1_ref[...], preferred_element_type=jnp.float32)
    v = jnp.dot(m2, w2_ref[...], preferred_element_type=jnp.float32)
    agg = (gc * t + (1.0 - gc) * v) * dis[:, None]
    acc = jnp.dot(agg, wua_ref[...], preferred_element_type=jnp.float32)
    acc += jnp.dot(h_ref[...], wuh_ref[...],
                   preferred_element_type=jnp.float32)
    o_ref[...] = jnp.maximum(acc + bu_ref[...], 0.0)


def kernel(H, edge_index, edge_attr, W_self, b_self, W1, W2, We1, be1, We2,
           be2, g, Wu, bu):
    N, D_IN = H.shape
    E, D_EDGE = edge_attr.shape
    D_OUT = Wu.shape[1]
    H_EDGE = We1.shape[1]
    deg_pad = NP

    BE = 16000
    edge_w = pl.pallas_call(
        _edge_mlp_body,
        out_shape=jax.ShapeDtypeStruct((E, 1), jnp.float32),
        grid=(E // BE,),
        in_specs=[
            pl.BlockSpec((BE, D_EDGE), lambda i: (i, 0)),
            pl.BlockSpec((D_EDGE, H_EDGE), lambda i: (0, 0)),
            pl.BlockSpec((H_EDGE,), lambda i: (0,)),
            pl.BlockSpec((H_EDGE, 1), lambda i: (0, 0)),
            pl.BlockSpec((1,), lambda i: (0,)),
        ],
        out_specs=pl.BlockSpec((BE, 1), lambda i: (i, 0)),
    )(edge_attr, We1, be1, We2, be2)

    pad = EP - E
    src2 = jnp.concatenate([edge_index[0],
                            jnp.zeros((pad,), jnp.int32)]).reshape(EP // C, C)
    dst2 = jnp.concatenate([edge_index[1],
                            jnp.zeros((pad,), jnp.int32)]).reshape(EP // C, C)
    w2 = jnp.concatenate([edge_w[:, 0],
                          jnp.zeros((pad,), jnp.float32)]).reshape(EP // C, C)

    srcd = src2.reshape(EP // CD, CD)
    dstd = dst2.reshape(EP // CD, CD)
    wd = w2.reshape(EP // CD, CD)
    degp = _make_deg_kernel(deg_pad)(srcd, dstd, wd)
    degt = degp[:, 0, :N].T

    BN = 1000
    hs = pl.pallas_call(
        _hs_body,
        out_shape=jax.ShapeDtypeStruct((N, D_IN), jnp.float32),
        grid=(N // BN,),
        in_specs=[
            pl.BlockSpec((BN, D_IN), lambda i: (i, 0)),
            pl.BlockSpec((BN, 2), lambda i: (i, 0)),
        ],
        out_specs=pl.BlockSpec((BN, D_IN), lambda i: (i, 0)),
    )(H, degt)

    spmm = _make_spmm_kernel(D_IN)

    mp = spmm(hs, src2, dst2, w2)

    xb = pl.pallas_call(
        _mid_body,
        out_shape=jax.ShapeDtypeStruct((N, D_IN), jnp.float32),
        grid=(N // BN,),
        in_specs=[
            pl.BlockSpec((NC, BN, D_IN), lambda i: (0, i, 0)),
            pl.BlockSpec((BN, 2), lambda i: (i, 0)),
        ],
        out_specs=pl.BlockSpec((BN, D_IN), lambda i: (i, 0)),
    )(mp, degt)

    m2p = spmm(xb, src2, dst2, w2)

    out = pl.pallas_call(
        _final_body,
        out_shape=jax.ShapeDtypeStruct((N, D_OUT), jnp.float32),
        grid=(N // BN,),
        in_specs=[
            pl.BlockSpec((NC, BN, D_IN), lambda i: (0, i, 0)),
            pl.BlockSpec((NC, BN, D_IN), lambda i: (0, i, 0)),
            pl.BlockSpec((BN, D_IN), lambda i: (i, 0)),
            pl.BlockSpec((BN, 2), lambda i: (i, 0)),
            pl.BlockSpec((1,), lambda i: (0,)),
            pl.BlockSpec((D_IN, D_OUT), lambda i: (0, 0)),
            pl.BlockSpec((D_IN, D_OUT), lambda i: (0, 0)),
            pl.BlockSpec((D_OUT, D_OUT), lambda i: (0, 0)),
            pl.BlockSpec((D_IN, D_OUT), lambda i: (0, 0)),
            pl.BlockSpec((D_OUT,), lambda i: (0,)),
        ],
        out_specs=pl.BlockSpec((BN, D_OUT), lambda i: (i, 0)),
    )(mp, m2p, H, degt, jnp.reshape(g, (1,)), W1, W2,
      Wu[:D_OUT], Wu[D_OUT:], bu)
    return out
